# baseline (speedup 1.0000x reference)
import functools

import jax
import jax.numpy as jnp
from jax import lax
from jax.experimental import pallas as pl
from jax.experimental.pallas import tpu as pltpu
from jax.experimental.pallas import tpu_sc as plsc

N_NODES = 10000
D_IN = 256
D_OUT = 256
N_TILES = 16
CHUNK = 128
ROWS_PER_TILE = 624
TAIL_ROWS = N_NODES - N_TILES * ROWS_PER_TILE
WPR = D_OUT // 2


def _mm_body(x_ref, w_ref, o_ref):
    o_ref[...] = lax.dot_general(
        x_ref[...], w_ref[...],
        dimension_numbers=(((1,), (1,)), ((), ())),
        preferred_element_type=jnp.float32,
    ).astype(jnp.bfloat16)


def _matmul(x, W):
    n = x.shape[0]
    blk = 400
    return pl.pallas_call(
        _mm_body,
        grid=(n // blk,),
        in_specs=[
            pl.BlockSpec((blk, D_IN), lambda i: (i, 0)),
            pl.BlockSpec((D_OUT, D_IN), lambda i: (0, 0)),
        ],
        out_specs=pl.BlockSpec((blk, D_OUT), lambda i: (i, 0)),
        out_shape=jax.ShapeDtypeStruct((n, D_OUT), jnp.bfloat16),
    )(x, W)


def _merge_body(a_ref, b_ref, o_ref):
    o_ref[...] = jnp.maximum(
        a_ref[0].astype(jnp.float32) + b_ref[0].astype(jnp.float32), 0.0)


def _merge_relu(acc2):
    n = acc2.shape[1]
    blk = 400
    return pl.pallas_call(
        _merge_body,
        grid=(n // blk,),
        in_specs=[
            pl.BlockSpec((1, blk, D_OUT), lambda i: (0, i, 0)),
            pl.BlockSpec((1, blk, D_OUT), lambda i: (1, i, 0)),
        ],
        out_specs=pl.BlockSpec((blk, D_OUT), lambda i: (i, 0)),
        out_shape=jax.ShapeDtypeStruct((n, D_OUT), jnp.float32),
    )(acc2, acc2)


def _sc_aggregate(h2, srcs, dsts, vals, zrows, nch):
    mesh = plsc.VectorSubcoreMesh(core_axis_name="c", subcore_axis_name="s")

    @functools.partial(
        pl.kernel,
        mesh=mesh,
        compiler_params=pltpu.CompilerParams(
            needs_layout_passes=False, use_tc_tiling_on_sc=False),
        out_type=jax.ShapeDtypeStruct((2, N_NODES, D_OUT), jnp.bfloat16),
        scratch_types=[
            pltpu.VMEM((nch, CHUNK), jnp.int32),
            pltpu.VMEM((nch, CHUNK), jnp.int32),
            pltpu.VMEM((nch, CHUNK), jnp.float32),
            pltpu.VMEM((CHUNK, D_OUT), jnp.bfloat16),
            pltpu.VMEM((CHUNK, D_OUT), jnp.bfloat16),
            pltpu.VMEM_SHARED((N_NODES, D_OUT), jnp.bfloat16),
            pltpu.SemaphoreType.DMA,
        ],
    )
    def body(h_ref, src_ref, dst_ref, val_ref, z_ref, out_ref,
             src_v, dst_v, val_v, gb, sb, acc_s, sem):
        c = lax.axis_index("c")
        s = lax.axis_index("s")

        sl_e = pl.ds((c * N_TILES + s) * nch, nch)
        pltpu.sync_copy(src_ref.at[sl_e], src_v)
        pltpu.sync_copy(dst_ref.at[sl_e], dst_v)
        pltpu.sync_copy(val_ref.at[sl_e], val_v)

        pltpu.sync_copy(z_ref, acc_s.at[pl.ds(s * ROWS_PER_TILE, ROWS_PER_TILE)])

        @pl.when(s == 0)
        def _():
            pltpu.sync_copy(
                z_ref.at[pl.ds(0, TAIL_ROWS)],
                acc_s.at[pl.ds(N_TILES * ROWS_PER_TILE, TAIL_ROWS)],
            )

        plsc.subcore_barrier()

        def scale(m):
            @plsc.parallel_loop(0, CHUNK // 16)
            def blk_body(b):
                vblk = val_v[m, pl.ds(b * 16, 16)]
                for k in range(16):
                    spl = jnp.full((16,), vblk[k], jnp.float32)
                    scal = plsc.pack(
                        spl, spl, format=plsc.PackFormat.INTERLEAVED)
                    e = b * 16 + k
                    for t in range(D_OUT // 32):
                        col = pl.ds(t * 32, 32)
                        sb[e, col] = gb[e, col] * scal

        def chunk_body(j, carry):
            pltpu.async_copy(h_ref.at[src_v.at[j]], gb, sem).wait()
            scale(j)
            pltpu.sync_copy(sb, acc_s.at[dst_v.at[j]], add=True)
            return carry

        lax.fori_loop(0, nch, chunk_body, 0)
        plsc.subcore_barrier()

        pltpu.sync_copy(
            acc_s.at[pl.ds(s * ROWS_PER_TILE, ROWS_PER_TILE)],
            out_ref.at[c, pl.ds(s * ROWS_PER_TILE, ROWS_PER_TILE)])

        @pl.when(s == 0)
        def _():
            pltpu.sync_copy(
                acc_s.at[pl.ds(N_TILES * ROWS_PER_TILE, TAIL_ROWS)],
                out_ref.at[c, pl.ds(N_TILES * ROWS_PER_TILE, TAIL_ROWS)])

    return body(h2, srcs, dsts, vals, zrows)


def kernel(x, W, adj_values, edge_index):
    n, e = x.shape[0], adj_values.shape[0]
    nch = -(-e // (2 * N_TILES * CHUNK))
    e_pad = nch * 2 * N_TILES * CHUNK
    pad = e_pad - e

    h2 = _matmul(x, W)

    srcs = jnp.pad(edge_index[1], (0, pad)).reshape(-1, CHUNK)
    dsts = jnp.pad(edge_index[0], (0, pad)).reshape(-1, CHUNK)
    vals = jnp.pad(adj_values, (0, pad)).reshape(-1, CHUNK)
    zrows = jnp.zeros((ROWS_PER_TILE, D_OUT), jnp.bfloat16)

    acc2 = _sc_aggregate(h2, srcs, dsts, vals, zrows, nch)
    return _merge_relu(acc2)


# speedup vs baseline: 1.1134x; 1.1134x over previous
import functools

import jax
import jax.numpy as jnp
from jax import lax
from jax.experimental import pallas as pl
from jax.experimental.pallas import tpu as pltpu
from jax.experimental.pallas import tpu_sc as plsc

N_NODES = 10000
D_IN = 256
D_OUT = 256
N_TILES = 16
CHUNK = 128
ROWS_PER_TILE = 624
TAIL_ROWS = N_NODES - N_TILES * ROWS_PER_TILE
WPR = D_OUT // 2


def _mm_body(x_ref, w_ref, o_ref):
    o_ref[...] = lax.dot_general(
        x_ref[...], w_ref[...],
        dimension_numbers=(((1,), (1,)), ((), ())),
        preferred_element_type=jnp.float32,
    ).astype(jnp.bfloat16)


def _matmul(x, W):
    n = x.shape[0]
    blk = 400
    return pl.pallas_call(
        _mm_body,
        grid=(n // blk,),
        in_specs=[
            pl.BlockSpec((blk, D_IN), lambda i: (i, 0)),
            pl.BlockSpec((D_OUT, D_IN), lambda i: (0, 0)),
        ],
        out_specs=pl.BlockSpec((blk, D_OUT), lambda i: (i, 0)),
        out_shape=jax.ShapeDtypeStruct((n, D_OUT), jnp.bfloat16),
    )(x, W)


def _merge_body(a_ref, b_ref, o_ref):
    o_ref[...] = jnp.maximum(
        a_ref[0].astype(jnp.float32) + b_ref[0].astype(jnp.float32), 0.0)


def _merge_relu(acc2):
    n = acc2.shape[1]
    blk = 400
    return pl.pallas_call(
        _merge_body,
        grid=(n // blk,),
        in_specs=[
            pl.BlockSpec((1, blk, D_OUT), lambda i: (0, i, 0)),
            pl.BlockSpec((1, blk, D_OUT), lambda i: (1, i, 0)),
        ],
        out_specs=pl.BlockSpec((blk, D_OUT), lambda i: (i, 0)),
        out_shape=jax.ShapeDtypeStruct((n, D_OUT), jnp.float32),
    )(acc2, acc2)


def _sc_aggregate(h2, edge_index, adj_values, zrows, ept):
    mesh = plsc.VectorSubcoreMesh(core_axis_name="c", subcore_axis_name="s")
    nch = -(-ept // CHUNK)
    cap = nch * CHUNK
    ntail = -(-(cap - ept) // 16)

    @functools.partial(
        pl.kernel,
        mesh=mesh,
        compiler_params=pltpu.CompilerParams(
            needs_layout_passes=False, use_tc_tiling_on_sc=False),
        out_type=jax.ShapeDtypeStruct((2, N_NODES, D_OUT), jnp.bfloat16),
        scratch_types=[
            pltpu.VMEM((cap + 16,), jnp.int32),
            pltpu.VMEM((cap + 16,), jnp.int32),
            pltpu.VMEM((cap + 16,), jnp.float32),
            pltpu.VMEM((CHUNK, D_OUT), jnp.bfloat16),
            pltpu.VMEM((CHUNK, D_OUT), jnp.bfloat16),
            pltpu.VMEM_SHARED((N_NODES, D_OUT), jnp.bfloat16),
            pltpu.SemaphoreType.DMA,
        ],
    )
    def body(h_ref, ei_ref, val_ref, z_ref, out_ref,
             src_v, dst_v, val_v, gb, sb, acc_s, sem):
        c = lax.axis_index("c")
        s = lax.axis_index("s")

        off = (c * N_TILES + s) * ept
        pltpu.sync_copy(ei_ref.at[1, pl.ds(off, ept)], src_v.at[pl.ds(0, ept)])
        pltpu.sync_copy(ei_ref.at[0, pl.ds(off, ept)], dst_v.at[pl.ds(0, ept)])
        pltpu.sync_copy(val_ref.at[pl.ds(off, ept)], val_v.at[pl.ds(0, ept)])
        zi = jnp.zeros((16,), jnp.int32)
        zf = jnp.zeros((16,), jnp.float32)
        for k in range(ntail):
            src_v[pl.ds(ept + k * 16, 16)] = zi
            dst_v[pl.ds(ept + k * 16, 16)] = zi
            val_v[pl.ds(ept + k * 16, 16)] = zf

        pltpu.sync_copy(z_ref, acc_s.at[pl.ds(s * ROWS_PER_TILE, ROWS_PER_TILE)])

        @pl.when(s == 0)
        def _():
            pltpu.sync_copy(
                z_ref.at[pl.ds(0, TAIL_ROWS)],
                acc_s.at[pl.ds(N_TILES * ROWS_PER_TILE, TAIL_ROWS)],
            )

        plsc.subcore_barrier()

        def scale(j):
            @plsc.parallel_loop(0, CHUNK // 16)
            def blk_body(b):
                vblk = val_v[pl.ds(j * CHUNK + b * 16, 16)]
                for k in range(16):
                    spl = jnp.full((16,), vblk[k], jnp.float32)
                    scal = plsc.pack(
                        spl, spl, format=plsc.PackFormat.INTERLEAVED)
                    e = b * 16 + k
                    for t in range(D_OUT // 32):
                        col = pl.ds(t * 32, 32)
                        sb[e, col] = gb[e, col] * scal

        def chunk_body(j, carry):
            sl_c = pl.ds(j * CHUNK, CHUNK)
            pltpu.async_copy(h_ref.at[src_v.at[sl_c]], gb, sem).wait()
            scale(j)
            pltpu.sync_copy(sb, acc_s.at[dst_v.at[sl_c]], add=True)
            return carry

        lax.fori_loop(0, nch, chunk_body, 0)
        plsc.subcore_barrier()

        pltpu.sync_copy(
            acc_s.at[pl.ds(s * ROWS_PER_TILE, ROWS_PER_TILE)],
            out_ref.at[c, pl.ds(s * ROWS_PER_TILE, ROWS_PER_TILE)])

        @pl.when(s == 0)
        def _():
            pltpu.sync_copy(
                acc_s.at[pl.ds(N_TILES * ROWS_PER_TILE, TAIL_ROWS)],
                out_ref.at[c, pl.ds(N_TILES * ROWS_PER_TILE, TAIL_ROWS)])

    return body(h2, edge_index, adj_values, zrows)


def kernel(x, W, adj_values, edge_index):
    n, e = x.shape[0], adj_values.shape[0]
    ept = e // (2 * N_TILES)

    h2 = _matmul(x, W)
    zrows = jnp.zeros((ROWS_PER_TILE, D_OUT), jnp.bfloat16)

    acc2 = _sc_aggregate(h2, edge_index, adj_values, zrows, ept)
    return _merge_relu(acc2)
